# baseline (speedup 1.0000x reference)
import jax
import jax.numpy as jnp
from jax.experimental import pallas as pl

_C = 768
_H = 12
_HD = 64
_BQ = 1024


def _attn_body(x_ref, w_ref, bq_ref, bv_ref, ao_ref):
    xb = x_ref[0]
    n = xb.shape[0]
    h = pl.program_id(1)
    qkv = jnp.dot(xb, w_ref[0], preferred_element_type=jnp.float32)
    q = (qkv[:, 0:_HD] + bq_ref[0, 0]) * (_HD ** -0.5)
    k = qkv[:, _HD:2 * _HD]
    v = qkv[:, 2 * _HD:3 * _HD]
    mq = jnp.max(jnp.sum(q * q, axis=-1))
    mk = jnp.max(jnp.sum(k * k, axis=-1))
    bound = jnp.sqrt(mq * mk)
    v_ext = jnp.concatenate([v, jnp.ones((n, 1), jnp.float32)], axis=-1)
    bv = bv_ref[0, 0]
    for i in range(n // _BQ):
        qi = q[i * _BQ:(i + 1) * _BQ]
        s = jax.lax.dot_general(qi, k, (((1,), (1,)), ((), ())),
                                preferred_element_type=jnp.float32)
        e = jnp.exp(s - bound)
        o = jnp.dot(e, v_ext, preferred_element_type=jnp.float32)
        res = o[:, :_HD] * (1.0 / o[:, _HD:_HD + 1]) + bv
        for j in range(4):
            @pl.when(h % 4 == j)
            def _(res=res, i=i, j=j):
                ao_ref[0, 0, i * _BQ:(i + 1) * _BQ,
                       j * _HD:(j + 1) * _HD] = res


def _proj_body(ao_ref, pw_ref, pb_ref, out_ref):
    acc = pb_ref[0] + jnp.zeros((ao_ref.shape[2], _C), jnp.float32)
    for j in range(3):
        acc = acc + jax.lax.dot_general(
            ao_ref[0, j], pw_ref[:, 4 * _HD * j:4 * _HD * (j + 1)],
            (((1,), (1,)), ((), ())), preferred_element_type=jnp.float32)
    out_ref[0] = acc


def kernel(x, xpos, qkv_w, qkv_b, proj_w, proj_b):
    del xpos
    B, N, C = x.shape
    w192 = jnp.transpose(qkv_w.reshape(3, _H, _HD, C), (1, 3, 0, 2)).reshape(_H, C, 3 * _HD)
    b3 = qkv_b.reshape(3 * _H, 1, _HD)
    pb2 = proj_b.reshape(1, C)

    attnout = pl.pallas_call(
        _attn_body,
        grid=(B, _H),
        in_specs=[
            pl.BlockSpec((1, N, C), lambda b, h: (b, 0, 0)),
            pl.BlockSpec((1, C, 3 * _HD), lambda b, h: (h, 0, 0)),
            pl.BlockSpec((1, 1, _HD), lambda b, h: (h, 0, 0)),
            pl.BlockSpec((1, 1, _HD), lambda b, h: (2 * _H + h, 0, 0)),
        ],
        out_specs=pl.BlockSpec((1, 1, N, 4 * _HD), lambda b, h: (b, h // 4, 0, 0)),
        out_shape=jax.ShapeDtypeStruct((B, _H // 4, N, 4 * _HD), jnp.float32),
    )(x, w192, b3, b3)

    out = pl.pallas_call(
        _proj_body,
        grid=(B,),
        in_specs=[
            pl.BlockSpec((1, _H // 4, N, 4 * _HD), lambda b: (b, 0, 0, 0)),
            pl.BlockSpec((C, C), lambda b: (0, 0)),
            pl.BlockSpec((1, C), lambda b: (0, 0)),
        ],
        out_specs=pl.BlockSpec((1, N, C), lambda b: (b, 0, 0)),
        out_shape=jax.ShapeDtypeStruct((B, N, C), jnp.float32),
    )(attnout, proj_w, pb2)
    return out


# speedup vs baseline: 1.3012x; 1.3012x over previous
import jax
import jax.numpy as jnp
from jax.experimental import pallas as pl
from jax.experimental.pallas import tpu as pltpu

_C = 768
_H = 12
_HD = 64
_G = 4
_BQ = 1024
_LOG2E = 1.4426950408889634


def _attn_body(x_ref, w_ref, bq_ref, bv_ref, pw_ref, pb_ref, out_ref):
    xb = x_ref[0]
    n = xb.shape[0]
    g = pl.program_id(1)
    qkv = jnp.dot(xb, w_ref[0], preferred_element_type=jnp.float32)
    q4 = (qkv[:, 0:_G * _HD] + bq_ref[0, 0]) * (_HD ** -0.5 * _LOG2E)
    k4 = qkv[:, _G * _HD:2 * _G * _HD]
    v4 = qkv[:, 2 * _G * _HD:3 * _G * _HD]
    rsel = jax.lax.broadcasted_iota(jnp.int32, (2 * _G * _HD, 2 * _G), 0) // _HD
    csel = jax.lax.broadcasted_iota(jnp.int32, (2 * _G * _HD, 2 * _G), 1)
    ones_bd = (rsel == csel).astype(jnp.float32)
    qk4 = jnp.concatenate([q4, k4], axis=-1)
    norms = jnp.max(jnp.dot(qk4 * qk4, ones_bd,
                            preferred_element_type=jnp.float32), axis=0)
    ones_col = jnp.ones((n, 1), jnp.float32)
    bv = bv_ref[0, 0]
    for i in range(n // _BQ):
        res4 = []
        for hh in range(_G):
            qh = q4[i * _BQ:(i + 1) * _BQ, hh * _HD:(hh + 1) * _HD]
            kh = k4[:, hh * _HD:(hh + 1) * _HD]
            vh = v4[:, hh * _HD:(hh + 1) * _HD]
            bound = jnp.sqrt(norms[hh] * norms[_G + hh])
            s = jax.lax.dot_general(qh, kh, (((1,), (1,)), ((), ())),
                                    preferred_element_type=jnp.float32)
            e = jnp.exp2(s - bound)
            o = jnp.dot(e, jnp.concatenate([vh, ones_col], axis=-1),
                        preferred_element_type=jnp.float32)
            res4.append(o[:, :_HD] * (1.0 / o[:, _HD:_HD + 1])
                        + bv[hh * _HD:(hh + 1) * _HD])
        part = jnp.dot(jnp.concatenate(res4, axis=-1), pw_ref[0],
                       preferred_element_type=jnp.float32)
        sl = pl.dslice(i * _BQ, _BQ)

        @pl.when(g == 0)
        def _():
            out_ref[0, sl] = part + pb_ref[0]

        @pl.when(g != 0)
        def _():
            out_ref[0, sl] += part


def kernel(x, xpos, qkv_w, qkv_b, proj_w, proj_b):
    del xpos
    B, N, C = x.shape
    w768 = jnp.transpose(qkv_w.reshape(3, 3, _G, _HD, C),
                         (1, 4, 0, 2, 3)).reshape(3, C, 3 * _G * _HD)
    b2 = qkv_b.reshape(3, 3, 1, _G * _HD)
    bqv = jnp.concatenate([b2[0], b2[2]], axis=0).reshape(6, 1, _G * _HD)
    pw3 = proj_w.T.reshape(3, _G * _HD, C)
    pb2 = proj_b.reshape(1, C)

    out = pl.pallas_call(
        _attn_body,
        grid=(B, 3),
        in_specs=[
            pl.BlockSpec((1, N, C), lambda b, g: (b, 0, 0)),
            pl.BlockSpec((1, C, 3 * _G * _HD), lambda b, g: (g, 0, 0)),
            pl.BlockSpec((1, 1, _G * _HD), lambda b, g: (g, 0, 0)),
            pl.BlockSpec((1, 1, _G * _HD), lambda b, g: (3 + g, 0, 0)),
            pl.BlockSpec((1, _G * _HD, C), lambda b, g: (g, 0, 0)),
            pl.BlockSpec((1, C), lambda b, g: (0, 0)),
        ],
        out_specs=pl.BlockSpec((1, N, C), lambda b, g: (b, 0, 0)),
        out_shape=jax.ShapeDtypeStruct((B, N, C), jnp.float32),
        compiler_params=pltpu.CompilerParams(
            dimension_semantics=("parallel", "arbitrary")),
    )(x, w768, bqv, bqv, pw3, pb2)
    return out
